# baseline (speedup 1.0000x reference)
import functools

import jax
import jax.numpy as jnp
from jax import lax
from jax.experimental import pallas as pl
from jax.experimental.pallas import tpu as pltpu
from jax.experimental.pallas import tpu_sc as plsc

L = 5
D = 128
NG = 64
N = 10000
E = 320000

NC = 2
NS = 16
NW = NC * NS
EPW = E // NW
C = 128
EPWP = 10240
PADW = EPWP - EPW
NCHUNK = EPWP // C
GS = 8
NGRP = NCHUNK // GS
NPAD = 10240
RPS = NPAD // NS


def _sc_segment_sum_edges(h, src3, dst3):
  mesh = plsc.VectorSubcoreMesh(core_axis_name="c", subcore_axis_name="s")

  @functools.partial(
      pl.kernel,
      mesh=mesh,
      out_type=jax.ShapeDtypeStruct((NC * NPAD, D), jnp.float32),
      scratch_types=[
          pltpu.VMEM((GS, C), jnp.int32),
          pltpu.VMEM((GS, C), jnp.int32),
          pltpu.VMEM((2, C, D), jnp.float32),
          pltpu.VMEM_SHARED((NPAD, D), jnp.float32),
          pltpu.SemaphoreType.DMA((2,)),
      ],
  )
  def k(h_hbm, src_hbm, dst_hbm, out_hbm, sidx, didx, rows_v, agg_sh, sem):
    cid = lax.axis_index("c")
    sid = lax.axis_index("s")
    wid = sid * NC + cid

    z16 = jnp.zeros((16,), jnp.float32)

    def zrow(r, carry):
      for j in range(D // 16):
        rows_v[0, r, pl.ds(j * 16, 16)] = z16
      return carry

    lax.fori_loop(0, C, zrow, 0)

    def zcp(kk, carry):
      pltpu.sync_copy(rows_v.at[0], agg_sh.at[pl.ds(sid * RPS + kk * C, C)])
      return carry

    lax.fori_loop(0, RPS // C, zcp, 0)
    plsc.subcore_barrier()

    def group(g, carry):
      pltpu.sync_copy(src_hbm.at[wid, pl.ds(g * GS, GS)], sidx)
      pltpu.sync_copy(dst_hbm.at[wid, pl.ds(g * GS, GS)], didx)
      for jj in range(GS):
        pltpu.async_copy(h_hbm.at[sidx.at[jj]], rows_v.at[0],
                         sem.at[0]).wait()
        pltpu.sync_copy(rows_v.at[0], agg_sh.at[didx.at[jj]], add=True)
      return carry

    lax.fori_loop(0, NGRP, group, 0)

    plsc.subcore_barrier()
    pltpu.sync_copy(agg_sh.at[pl.ds(sid * RPS, RPS)],
                    out_hbm.at[pl.ds(cid * NPAD + sid * RPS, RPS)])

  return k(h, src3, dst3)


def _dot(a, b):
  return jnp.dot(a, b, preferred_element_type=jnp.float32,
                 precision=lax.Precision.HIGHEST)


def _dotw(a, b):
  return jnp.dot(a.astype(jnp.bfloat16), b.astype(jnp.bfloat16),
                 preferred_element_type=jnp.float32)


def _bn_rows(z, g, b):
  mu = jnp.mean(z, axis=0, keepdims=True)
  var = jnp.mean((z - mu) ** 2, axis=0, keepdims=True)
  return (z - mu) / jnp.sqrt(var + 1e-5) * g + b


def _encoder_call(x, W_enc, b_enc, vn_w, bcol, brow):
  def body(x_ref, w_ref, b_ref, v_ref, bc_ref, br_ref, h_ref, oh_ref, ot_ref):
    h_ref[...] = (_dotw(x_ref[...], w_ref[...]) + b_ref[...] + v_ref[...])
    oh_ref[...] = (bc_ref[...] == lax.broadcasted_iota(
        jnp.int32, (N, NG), 1)).astype(jnp.float32)
    ot_ref[...] = (br_ref[...] == lax.broadcasted_iota(
        jnp.int32, (NG, N), 0)).astype(jnp.float32)

  return pl.pallas_call(
      body,
      out_shape=(jax.ShapeDtypeStruct((N, D), jnp.float32),
                 jax.ShapeDtypeStruct((N, NG), jnp.float32),
                 jax.ShapeDtypeStruct((NG, N), jnp.float32)),
  )(x, W_enc, b_enc.reshape(1, D), vn_w.reshape(1, D), bcol, brow)


def _layer_call(hpre, agg0, agg1, oneh, oneT, vn, W1, b1, W2, b2, g, b,
                vW1, vb1, vg1, vB1, vW2, vb2, vg2, vB2):
  def body(hp, a0, a1, oh, ot, vnr, W1r, b1r, W2r, b2r, gr, br,
           vW1r, vb1r, vg1r, vB1r, vW2r, vb2r, vg2r, vB2r, hnext, vnnext):
    z = hp[...] + (a0[...] + a1[...])
    z = jnp.maximum(_dotw(z, W1r[...]) + b1r[...], 0.0)
    z = _dotw(z, W2r[...]) + b2r[...]
    z = _bn_rows(z, gr[...], br[...])
    h = jnp.where(z > 0, z, 0.1 * z)
    t = _dot(ot[...], h) + vnr[...]
    u = _dotw(t, vW1r[...]) + vb1r[...]
    u = jnp.maximum(_bn_rows(u, vg1r[...], vB1r[...]), 0.0)
    u = _dotw(u, vW2r[...]) + vb2r[...]
    u = jnp.maximum(_bn_rows(u, vg2r[...], vB2r[...]), 0.0)
    vnnext[...] = u
    hnext[...] = h + _dot(oh[...], u)

  return pl.pallas_call(
      body,
      out_shape=(jax.ShapeDtypeStruct((N, D), jnp.float32),
                 jax.ShapeDtypeStruct((NG, D), jnp.float32)),
  )(hpre, agg0, agg1, oneh, oneT, vn,
    W1, b1.reshape(1, D), W2, b2.reshape(1, D),
    g.reshape(1, D), b.reshape(1, D),
    vW1, vb1.reshape(1, D), vg1.reshape(1, D), vB1.reshape(1, D),
    vW2, vb2.reshape(1, D), vg2.reshape(1, D), vB2.reshape(1, D))


def _final_call(hpre, agg0, agg1, oneT, W1, b1, W2, b2, g, b,
                hW1, hb1, hW2, hb2, hW3, hb3):
  def body(hp, a0, a1, ot, W1r, b1r, W2r, b2r, gr, br,
           hW1r, hb1r, hW2r, hb2r, hW3r, hb3r, out):
    z = hp[...] + (a0[...] + a1[...])
    z = jnp.maximum(_dotw(z, W1r[...]) + b1r[...], 0.0)
    z = _dotw(z, W2r[...]) + b2r[...]
    z = _bn_rows(z, gr[...], br[...])
    h = jnp.where(z > 0, z, 0.1 * z)
    gpool = _dot(ot[...], h)
    gpool = jnp.maximum(_dotw(gpool, hW1r[...]) + hb1r[...], 0.0)
    gpool = jnp.maximum(_dotw(gpool, hW2r[...]) + hb2r[...], 0.0)
    out[...] = _dotw(gpool, hW3r[...]) + hb3r[...]

  return pl.pallas_call(
      body,
      out_shape=jax.ShapeDtypeStruct((NG, 11), jnp.float32),
  )(hpre, agg0, agg1, oneT,
    W1, b1.reshape(1, D), W2, b2.reshape(1, D),
    g.reshape(1, D), b.reshape(1, D),
    hW1, hb1.reshape(1, D // 2), hW2, hb2.reshape(1, D // 4),
    hW3, hb3.reshape(1, 11))


def kernel(x, edge_index, batch, W_enc, b_enc, vn_w, gin_W1, gin_b1, gin_W2,
           gin_b2, bn_g, bn_b, vn_W1, vn_b1, vn_g1, vn_beta1, vn_W2, vn_b2,
           vn_g2, vn_beta2, hW1, hb1, hW2, hb2, hW3, hb3):
  src = jnp.pad(edge_index[0].reshape(NW, EPW),
                ((0, 0), (0, PADW))).reshape(NW, NCHUNK, C)
  dst = jnp.pad(edge_index[1].reshape(NW, EPW), ((0, 0), (0, PADW)),
                constant_values=N).reshape(NW, NCHUNK, C)
  bcol = batch.reshape(N, 1)
  brow = batch.reshape(1, N)

  hpre, oneh, oneT = _encoder_call(x, W_enc, b_enc, vn_w, bcol, brow)
  vn = jnp.broadcast_to(vn_w.reshape(1, D), (NG, D))

  for l in range(L):
    agg = _sc_segment_sum_edges(hpre, src, dst)
    a0 = agg[:N]
    a1 = agg[NPAD:NPAD + N]
    if l < L - 1:
      hpre, vn = _layer_call(
          hpre, a0, a1, oneh, oneT, vn,
          gin_W1[l], gin_b1[l], gin_W2[l], gin_b2[l], bn_g[l], bn_b[l],
          vn_W1[l], vn_b1[l], vn_g1[l], vn_beta1[l],
          vn_W2[l], vn_b2[l], vn_g2[l], vn_beta2[l])
    else:
      out = _final_call(
          hpre, a0, a1, oneT,
          gin_W1[l], gin_b1[l], gin_W2[l], gin_b2[l], bn_g[l], bn_b[l],
          hW1, hb1, hW2, hb2, hW3, hb3)
  return out


# speedup vs baseline: 2.6622x; 2.6622x over previous
import functools

import jax
import jax.numpy as jnp
from jax import lax
from jax.experimental import pallas as pl
from jax.experimental.pallas import tpu as pltpu
from jax.experimental.pallas import tpu_sc as plsc

L = 5
D = 128
NG = 64
N = 10000
E = 320000

NC = 2
NS = 16
NW = NC * NS
EPW = E // NW
C = 80
NCHUNK = EPW // C
NPAIR = (NCHUNK - 1) // 2
NPAD = 10240
RPS = NPAD // NS


def _sc_segment_sum_edges(h, src3, dst3):
  mesh = plsc.VectorSubcoreMesh(core_axis_name="c", subcore_axis_name="s")

  @functools.partial(
      pl.kernel,
      mesh=mesh,
      out_type=jax.ShapeDtypeStruct((NC * NPAD, D), jnp.float32),
      scratch_types=[
          pltpu.VMEM((C,), jnp.int32),
          pltpu.VMEM((C,), jnp.int32),
          pltpu.VMEM((C,), jnp.int32),
          pltpu.VMEM((C,), jnp.int32),
          pltpu.VMEM((C, D), jnp.float32),
          pltpu.VMEM((C, D), jnp.float32),
          pltpu.VMEM_SHARED((NPAD, D), jnp.float32),
          pltpu.SemaphoreType.DMA((6,)),
      ],
  )
  def k(h_hbm, src_hbm, dst_hbm, out_hbm, srcA, dstA, srcB, dstB,
        rowsA, rowsB, agg_sh, sem):
    cid = lax.axis_index("c")
    sid = lax.axis_index("s")
    wid = sid * NC + cid

    z16 = jnp.zeros((16,), jnp.float32)

    def zrow(r, carry):
      for j in range(D // 16):
        rowsA[r, pl.ds(j * 16, 16)] = z16
      return carry

    lax.fori_loop(0, C, zrow, 0)

    def zcp(kk, carry):
      pltpu.sync_copy(rowsA, agg_sh.at[pl.ds(sid * RPS + kk * C, C)])
      return carry

    lax.fori_loop(0, RPS // C, zcp, 0)
    plsc.subcore_barrier()

    base = wid * EPW

    def _idx(buf, sl, off):
      return pltpu.async_copy(buf.at[pl.ds(off, C)], sl[0], sem.at[sl[1]])

    pltpu.sync_copy(src_hbm.at[pl.ds(base, C)], srcA)
    pltpu.sync_copy(dst_hbm.at[pl.ds(base, C)], dstA)
    pltpu.sync_copy(src_hbm.at[pl.ds(base + C, C)], srcB)
    pltpu.sync_copy(dst_hbm.at[pl.ds(base + C, C)], dstB)
    pltpu.async_copy(h_hbm.at[srcA], rowsA, sem.at[0])
    pltpu.async_copy(h_hbm.at[srcB], rowsB, sem.at[1])

    def body(i, carry):
      j0 = 2 * i
      j1 = j0 + 1
      offA = base + (j0 + 2) * C
      offB = base + (j1 + 2) * C
      pltpu.make_async_copy(h_hbm.at[srcA], rowsA, sem.at[0]).wait()
      pltpu.sync_copy(rowsA, agg_sh.at[dstA], add=True)
      _idx(src_hbm, (srcA, 2), offA)
      _idx(dst_hbm, (dstA, 3), offA)
      pltpu.make_async_copy(h_hbm.at[srcB], rowsB, sem.at[1]).wait()
      pltpu.sync_copy(rowsB, agg_sh.at[dstB], add=True)

      @pl.when(j1 + 2 < NCHUNK)
      def _():
        _idx(src_hbm, (srcB, 4), offB)
        _idx(dst_hbm, (dstB, 5), offB)

      pltpu.make_async_copy(src_hbm.at[pl.ds(offA, C)], srcA,
                            sem.at[2]).wait()
      pltpu.make_async_copy(dst_hbm.at[pl.ds(offA, C)], dstA,
                            sem.at[3]).wait()
      pltpu.async_copy(h_hbm.at[srcA], rowsA, sem.at[0])

      @pl.when(j1 + 2 < NCHUNK)
      def _():
        pltpu.make_async_copy(src_hbm.at[pl.ds(offB, C)], srcB,
                              sem.at[4]).wait()
        pltpu.make_async_copy(dst_hbm.at[pl.ds(offB, C)], dstB,
                              sem.at[5]).wait()
        pltpu.async_copy(h_hbm.at[srcB], rowsB, sem.at[1])

      return carry

    lax.fori_loop(0, NPAIR, body, 0)

    pltpu.make_async_copy(h_hbm.at[srcA], rowsA, sem.at[0]).wait()
    pltpu.sync_copy(rowsA, agg_sh.at[dstA], add=True)

    plsc.subcore_barrier()
    pltpu.sync_copy(agg_sh.at[pl.ds(sid * RPS, RPS)],
                    out_hbm.at[pl.ds(cid * NPAD + sid * RPS, RPS)])

  return k(h, src3, dst3)


def _dot(a, b):
  return jnp.dot(a, b, preferred_element_type=jnp.float32,
                 precision=lax.Precision.HIGHEST)


def _dotw(a, b):
  return jnp.dot(a.astype(jnp.bfloat16), b.astype(jnp.bfloat16),
                 preferred_element_type=jnp.float32)


def _bn_rows(z, g, b):
  mu = jnp.mean(z, axis=0, keepdims=True)
  var = jnp.mean((z - mu) ** 2, axis=0, keepdims=True)
  return (z - mu) / jnp.sqrt(var + 1e-5) * g + b


def _encoder_call(x, W_enc, b_enc, vn_w, bcol, brow):
  def body(x_ref, w_ref, b_ref, v_ref, bc_ref, br_ref, h_ref, oh_ref, ot_ref):
    h_ref[...] = (_dotw(x_ref[...], w_ref[...]) + b_ref[...] + v_ref[...])
    oh_ref[...] = (bc_ref[...] == lax.broadcasted_iota(
        jnp.int32, (N, NG), 1)).astype(jnp.float32)
    ot_ref[...] = (br_ref[...] == lax.broadcasted_iota(
        jnp.int32, (NG, N), 0)).astype(jnp.float32)

  return pl.pallas_call(
      body,
      out_shape=(jax.ShapeDtypeStruct((N, D), jnp.float32),
                 jax.ShapeDtypeStruct((N, NG), jnp.float32),
                 jax.ShapeDtypeStruct((NG, N), jnp.float32)),
  )(x, W_enc, b_enc.reshape(1, D), vn_w.reshape(1, D), bcol, brow)


def _layer_call(hpre, agg0, agg1, oneh, oneT, vn, W1, b1, W2, b2, g, b,
                vW1, vb1, vg1, vB1, vW2, vb2, vg2, vB2):
  def body(hp, a0, a1, oh, ot, vnr, W1r, b1r, W2r, b2r, gr, br,
           vW1r, vb1r, vg1r, vB1r, vW2r, vb2r, vg2r, vB2r, hnext, vnnext):
    z = hp[...] + (a0[...] + a1[...])
    z = jnp.maximum(_dotw(z, W1r[...]) + b1r[...], 0.0)
    z = _dotw(z, W2r[...]) + b2r[...]
    z = _bn_rows(z, gr[...], br[...])
    h = jnp.where(z > 0, z, 0.1 * z)
    t = _dot(ot[...], h) + vnr[...]
    u = _dotw(t, vW1r[...]) + vb1r[...]
    u = jnp.maximum(_bn_rows(u, vg1r[...], vB1r[...]), 0.0)
    u = _dotw(u, vW2r[...]) + vb2r[...]
    u = jnp.maximum(_bn_rows(u, vg2r[...], vB2r[...]), 0.0)
    vnnext[...] = u
    hnext[...] = h + _dot(oh[...], u)

  return pl.pallas_call(
      body,
      out_shape=(jax.ShapeDtypeStruct((N, D), jnp.float32),
                 jax.ShapeDtypeStruct((NG, D), jnp.float32)),
  )(hpre, agg0, agg1, oneh, oneT, vn,
    W1, b1.reshape(1, D), W2, b2.reshape(1, D),
    g.reshape(1, D), b.reshape(1, D),
    vW1, vb1.reshape(1, D), vg1.reshape(1, D), vB1.reshape(1, D),
    vW2, vb2.reshape(1, D), vg2.reshape(1, D), vB2.reshape(1, D))


def _final_call(hpre, agg0, agg1, oneT, W1, b1, W2, b2, g, b,
                hW1, hb1, hW2, hb2, hW3, hb3):
  def body(hp, a0, a1, ot, W1r, b1r, W2r, b2r, gr, br,
           hW1r, hb1r, hW2r, hb2r, hW3r, hb3r, out):
    z = hp[...] + (a0[...] + a1[...])
    z = jnp.maximum(_dotw(z, W1r[...]) + b1r[...], 0.0)
    z = _dotw(z, W2r[...]) + b2r[...]
    z = _bn_rows(z, gr[...], br[...])
    h = jnp.where(z > 0, z, 0.1 * z)
    gpool = _dot(ot[...], h)
    gpool = jnp.maximum(_dotw(gpool, hW1r[...]) + hb1r[...], 0.0)
    gpool = jnp.maximum(_dotw(gpool, hW2r[...]) + hb2r[...], 0.0)
    out[...] = _dotw(gpool, hW3r[...]) + hb3r[...]

  return pl.pallas_call(
      body,
      out_shape=jax.ShapeDtypeStruct((NG, 11), jnp.float32),
  )(hpre, agg0, agg1, oneT,
    W1, b1.reshape(1, D), W2, b2.reshape(1, D),
    g.reshape(1, D), b.reshape(1, D),
    hW1, hb1.reshape(1, D // 2), hW2, hb2.reshape(1, D // 4),
    hW3, hb3.reshape(1, 11))


def kernel(x, edge_index, batch, W_enc, b_enc, vn_w, gin_W1, gin_b1, gin_W2,
           gin_b2, bn_g, bn_b, vn_W1, vn_b1, vn_g1, vn_beta1, vn_W2, vn_b2,
           vn_g2, vn_beta2, hW1, hb1, hW2, hb2, hW3, hb3):
  src = edge_index[0]
  dst = edge_index[1]
  bcol = batch.reshape(N, 1)
  brow = batch.reshape(1, N)

  hpre, oneh, oneT = _encoder_call(x, W_enc, b_enc, vn_w, bcol, brow)
  vn = jnp.broadcast_to(vn_w.reshape(1, D), (NG, D))

  for l in range(L):
    agg = _sc_segment_sum_edges(hpre, src, dst)
    a0 = agg[:N]
    a1 = agg[NPAD:NPAD + N]
    if l < L - 1:
      hpre, vn = _layer_call(
          hpre, a0, a1, oneh, oneT, vn,
          gin_W1[l], gin_b1[l], gin_W2[l], gin_b2[l], bn_g[l], bn_b[l],
          vn_W1[l], vn_b1[l], vn_g1[l], vn_beta1[l],
          vn_W2[l], vn_b2[l], vn_g2[l], vn_beta2[l])
    else:
      out = _final_call(
          hpre, a0, a1, oneT,
          gin_W1[l], gin_b1[l], gin_W2[l], gin_b2[l], bn_g[l], bn_b[l],
          hW1, hb1, hW2, hb2, hW3, hb3)
  return out
